# baseline (speedup 1.0000x reference)
import functools

import jax
import jax.numpy as jnp
from jax import lax
from jax.experimental import pallas as pl
from jax.experimental.pallas import tpu as pltpu
from jax.experimental.pallas import tpu_sc as plsc

BB = 256
HB = 128
L = 16


def kernel(lookup, table):
    B, T = lookup.shape
    V, D = table.shape
    DP = 128

    info = plsc.get_sparse_core_info()
    NW = info.num_cores * info.num_subcores
    NBLK = B // BB
    upw = (T * NBLK) // NW

    idx_flat = jnp.transpose(lookup).reshape(-1).astype(jnp.int32)
    table_p = jnp.pad(table, ((0, 0), (0, DP - D)))

    mesh = plsc.VectorSubcoreMesh(core_axis_name="c", subcore_axis_name="s")

    @functools.partial(
        pl.kernel,
        mesh=mesh,
        out_type=jax.ShapeDtypeStruct((T, D, B), jnp.float32),
        scratch_types=[
            *[pltpu.VMEM((BB,), jnp.int32) for _ in range(3)],
            *[pltpu.VMEM((BB, DP), jnp.float32) for _ in range(3)],
            *[pltpu.VMEM((D, HB), jnp.float32) for _ in range(3)],
            *[pltpu.SemaphoreType.DMA for _ in range(3)],
            *[pltpu.SemaphoreType.DMA for _ in range(3)],
            *[pltpu.SemaphoreType.DMA for _ in range(3)],
        ],
        compiler_params=pltpu.CompilerParams(
            use_tc_tiling_on_sc=True, needs_layout_passes=False
        ),
    )
    def gather_kernel(table_hbm, idx_hbm, out_hbm, *bufs):
        idxq = bufs[0:3]
        rows = bufs[3:6]
        obufh = bufs[6:9]
        isem = bufs[9:12]
        gsem = bufs[12:15]
        osem = bufs[15:18]

        wid = lax.axis_index("s") * info.num_cores + lax.axis_index("c")
        blk = wid // 2
        t_base = (wid % 2) * upw
        b0 = blk * BB

        iot = lax.iota(jnp.int32, L)
        pvec = [(iot + k) & (L - 1) for k in range(L)]

        def start_idx(u, q):
            t = t_base + u
            pltpu.async_copy(
                idx_hbm.at[pl.ds(t * B + b0, BB)], idxq[q], isem[q]
            )

        def wait_idx(q):
            pltpu.make_async_copy(
                idx_hbm.at[pl.ds(0, BB)], idxq[q], isem[q]
            ).wait()

        def fire_g(q, pr):
            for c in range(BB // 128):
                pltpu.async_copy(
                    table_hbm.at[idxq[q].at[pl.ds(c * 128, 128)]],
                    rows[pr].at[pl.ds(c * 128, 128)],
                    gsem[pr],
                )

        def drain_gather(pr):
            pltpu.make_async_copy(
                table_hbm.at[pl.ds(0, BB)], rows[pr], gsem[pr]
            ).wait()

        def transpose_half(pr, h, half):
            def bbody(i, _):
                bb = lax.shift_right_logical(i, 2)
                c = lax.bitwise_and(i, 3)
                rloc = bb * L + iot
                rread = rloc + half * HB
                cbase = c * L
                for k in range(L):
                    dcol = pvec[k] + cbase
                    v = plsc.load_gather(rows[pr], [rread, dcol])
                    plsc.store_scatter(obufh[h], [dcol, rloc], v)
                return _
            lax.fori_loop(0, (HB // L) * (D // L), bbody, None)

        def start_wb(u, h, half):
            t = t_base + u
            pltpu.async_copy(
                obufh[h], out_hbm.at[t, :, pl.ds(b0 + half * HB, HB)],
                osem[h],
            )

        def wait_wb(h):
            pltpu.make_async_copy(
                out_hbm.at[0, :, pl.ds(0, HB)], obufh[h], osem[h]
            ).wait()

        def unit_ops(u, pr, do_fire, qf, do_sidx, wA, hA, wB, hB):
            if do_fire:
                wait_idx(qf)
                fire_g(qf, (pr + 2) % 3)
            drain_gather(pr)
            if do_sidx:
                start_idx(u + 3, pr)
            if wA:
                wait_wb(hA)
            transpose_half(pr, hA, 0)
            start_wb(u, hA, 0)
            if wB:
                wait_wb(hB)
            transpose_half(pr, hB, 1)
            start_wb(u, hB, 1)

        start_idx(0, 0)
        start_idx(1, 1)
        start_idx(2, 2)
        wait_idx(0)
        fire_g(0, 0)
        wait_idx(1)
        fire_g(1, 1)

        unit_ops(0, 0, True, 2, True, False, 0, False, 1)
        unit_ops(1, 1, True, 0, True, False, 2, True, 0)

        def body(m, _):
            u0 = 2 + 3 * m
            for s in range(3):
                u = u0 + s
                pr = (2 + s) % 3
                unit_ops(u, pr, True, (4 + s) % 3, True,
                         True, (4 + 2 * s) % 3, True, (5 + 2 * s) % 3)
            return _

        lax.fori_loop(0, 31, body, None)

        unit_ops(95, 95 % 3, True, 97 % 3, True,
                 True, 190 % 3, True, 191 % 3)
        unit_ops(96, 96 % 3, True, 98 % 3, True,
                 True, 192 % 3, True, 193 % 3)
        unit_ops(97, 97 % 3, True, 99 % 3, False,
                 True, 194 % 3, True, 195 % 3)
        unit_ops(98, 98 % 3, False, 0, False,
                 True, 196 % 3, True, 197 % 3)
        unit_ops(99, 99 % 3, False, 0, False,
                 True, 198 % 3, True, 199 % 3)
        for h in range(3):
            wait_wb(h)

    out = gather_kernel(table_p, idx_flat)
    return jnp.transpose(out, (2, 0, 1))


# speedup vs baseline: 1.0267x; 1.0267x over previous
import functools

import jax
import jax.numpy as jnp
from jax import lax
from jax.experimental import pallas as pl
from jax.experimental.pallas import tpu as pltpu
from jax.experimental.pallas import tpu_sc as plsc

BB = 256
L = 16
NI = 4


def kernel(lookup, table):
    B, T = lookup.shape
    V, D = table.shape
    DP = 128

    info = plsc.get_sparse_core_info()
    NW = info.num_cores * info.num_subcores
    NBLK = B // BB
    upw = (T * NBLK) // NW

    idx_flat = jnp.transpose(lookup).reshape(-1).astype(jnp.int32)
    table_p = jnp.pad(table, ((0, 0), (0, DP - D)))

    mesh = plsc.VectorSubcoreMesh(core_axis_name="c", subcore_axis_name="s")

    @functools.partial(
        pl.kernel,
        mesh=mesh,
        out_type=jax.ShapeDtypeStruct((T, D, B), jnp.float32),
        scratch_types=[
            *[pltpu.VMEM((BB,), jnp.int32) for _ in range(NI)],
            *[pltpu.VMEM((BB, DP), jnp.float32) for _ in range(2)],
            *[pltpu.VMEM((D, BB), jnp.float32) for _ in range(2)],
            *[pltpu.SemaphoreType.DMA for _ in range(NI)],
            *[pltpu.SemaphoreType.DMA for _ in range(4)],
        ],
        compiler_params=pltpu.CompilerParams(
            use_tc_tiling_on_sc=True, needs_layout_passes=False
        ),
    )
    def gather_kernel(table_hbm, idx_hbm, out_hbm, *bufs):
        idxq = bufs[0:NI]
        rows = bufs[NI:NI + 2]
        obuf = bufs[NI + 2:NI + 4]
        isem = bufs[NI + 4:2 * NI + 4]
        gsem = bufs[2 * NI + 4:2 * NI + 6]
        osem = bufs[2 * NI + 6:2 * NI + 8]

        wid = lax.axis_index("s") * info.num_cores + lax.axis_index("c")
        blk = wid // 2
        t_base = (wid % 2) * upw
        b0 = blk * BB

        iot = lax.iota(jnp.int32, L)
        pvec = [(iot + k) & (L - 1) for k in range(L)]

        def start_idx(u, q):
            t = t_base + u
            pltpu.async_copy(
                idx_hbm.at[pl.ds(t * B + b0, BB)], idxq[q], isem[q]
            )

        def wait_idx(q):
            pltpu.make_async_copy(
                idx_hbm.at[pl.ds(0, BB)], idxq[q], isem[q]
            ).wait()

        def fire_g(q, p):
            for c in range(BB // 128):
                pltpu.async_copy(
                    table_hbm.at[idxq[q].at[pl.ds(c * 128, 128)]],
                    rows[p].at[pl.ds(c * 128, 128)],
                    gsem[p],
                )

        def drain_gather(p):
            pltpu.make_async_copy(
                table_hbm.at[pl.ds(0, BB)], rows[p], gsem[p]
            ).wait()

        def transpose(p):
            def bbody(bb, _):
                for half in range(2):
                    rrow = (bb * 2 + half) * L + iot
                    for c in range(D // L):
                        for k in range(L):
                            dcol = pvec[k] + c * L
                            v = plsc.load_gather(rows[p], [rrow, dcol])
                            plsc.store_scatter(obuf[p], [dcol, rrow], v)
                return _
            lax.fori_loop(0, BB // L // 2, bbody, None)

        def start_wb(u, p):
            t = t_base + u
            pltpu.async_copy(
                obuf[p], out_hbm.at[t, :, pl.ds(b0, BB)], osem[p]
            )

        def wait_wb(p):
            pltpu.make_async_copy(
                out_hbm.at[0, :, pl.ds(0, BB)], obuf[p], osem[p]
            ).wait()

        def unit_ops(u, p, do_start_idx, do_fire, do_wait_wb, q2, q1):
            if do_start_idx:
                start_idx(u + 2, q2)
            if do_fire:
                wait_idx(q1)
                fire_g(q1, 1 - p)
            drain_gather(p)
            if do_wait_wb:
                wait_wb(p)
            transpose(p)
            start_wb(u, p)

        start_idx(0, 0)
        start_idx(1, 1)
        wait_idx(0)
        fire_g(0, 0)
        unit_ops(0, 0, True, True, False, 2, 1)
        unit_ops(1, 1, True, True, False, 3, 2)

        def body(m, _):
            u0 = 2 + 4 * m
            for s in range(4):
                unit_ops(u0 + s, s % 2, True, True, True, s, (3 + s) % 4)
            return _

        lax.fori_loop(0, (upw - 4) // 4, body, None)

        unit_ops(upw - 2, 0, False, True, True, 0, (upw - 1) % NI)
        unit_ops(upw - 1, 1, False, False, True, 0, 0)
        wait_wb(0)
        wait_wb(1)

    out = gather_kernel(table_p, idx_flat)
    return jnp.transpose(out, (2, 0, 1))
